# baseline (speedup 1.0000x reference)
import functools

import jax
import jax.numpy as jnp
from jax import lax
from jax.experimental import pallas as pl
from jax.experimental.pallas import tpu as pltpu
from jax.experimental.pallas import tpu_sc as plsc

_NTOK = 8192
_TOPK = 2
_E = 8
_INTER = 4096
_HID = 1024
_NSLOT = _NTOK * _TOPK

_BM = 512
_NTILES = _NSLOT // _BM
_MAX_STEPS = _NTILES + _E - 1

_NC, _NS = 2, 16
_NW = _NC * _NS


def _make_ring_gather(nrows, width, ch, dtype=jnp.float32):
    rpw = nrows // _NW
    nch = rpw // ch
    assert rpw % ch == 0 and nch % 2 == 0

    def body(src_hbm, idx_hbm, out_hbm, idx_v, buf0, buf1,
             gsem0, gsem1, ssem0, ssem1):
        bufs, gsems, ssems = (buf0, buf1), (gsem0, gsem1), (ssem0, ssem1)
        wid = lax.axis_index("s") * _NC + lax.axis_index("c")
        base = wid * rpw
        pltpu.sync_copy(idx_hbm.at[pl.ds(base, rpw)], idx_v)

        def gather(c, b):
            pltpu.make_async_copy(
                src_hbm.at[idx_v.at[pl.ds(c * ch, ch)]], bufs[b], gsems[b]
            ).start()

        gather(0, 0)
        gather(1, 1)

        def outer(c0, carry):
            for b in (0, 1):
                c = 2 * c0 + b
                pltpu.make_async_copy(
                    src_hbm.at[idx_v.at[pl.ds(c * ch, ch)]], bufs[b], gsems[b]
                ).wait()
                scat = pltpu.make_async_copy(
                    bufs[b], out_hbm.at[pl.ds(base + c * ch, ch)], ssems[b])
                scat.start()
                scat.wait()

                @pl.when(c + 2 < nch)
                def _():
                    gather(c + 2, b)
            return carry

        lax.fori_loop(0, nch // 2, outer, 0)

    @functools.cache
    def build():
        return pl.kernel(
            body,
            mesh=plsc.VectorSubcoreMesh(
                core_axis_name="c", subcore_axis_name="s"),
            out_type=jax.ShapeDtypeStruct((nrows, width), dtype),
            scratch_types=[
                pltpu.VMEM((rpw,), jnp.int32),
                pltpu.VMEM((ch, width), dtype),
                pltpu.VMEM((ch, width), dtype),
                pltpu.SemaphoreType.DMA,
                pltpu.SemaphoreType.DMA,
                pltpu.SemaphoreType.DMA,
                pltpu.SemaphoreType.DMA,
            ],
        )

    return build


_NSPLIT = 1
_HROWS = _NSLOT // _NSPLIT
_HTILES = _HROWS // _BM
_MAX_STEPS_H = _HTILES + _E - 1

_sc_gather_xh = _make_ring_gather(_HROWS, _INTER, 8)
_sc_unsort_y = _make_ring_gather(_NSLOT, _HID // 2, 32, jnp.int32)


def _gemm_body(sm_ref, se_ref, lo_ref, hi_ref, xs_ref, w_ref, out_ref):
    i = pl.program_id(0)
    acc = jnp.dot(xs_ref[...].astype(jnp.bfloat16),
                  w_ref[...].astype(jnp.bfloat16),
                  preferred_element_type=jnp.float32)
    rows = lax.broadcasted_iota(jnp.int32, (_BM, _HID), 0)
    mask = (rows >= lo_ref[i]) & (rows < hi_ref[i])
    out_ref[...] = jnp.where(mask, acc.astype(jnp.bfloat16), out_ref[...])


def _gemm_body_alias(sm_ref, se_ref, lo_ref, hi_ref, xs_ref, w_ref, prev_ref,
                     out_ref):
    del prev_ref
    _gemm_body(sm_ref, se_ref, lo_ref, hi_ref, xs_ref, w_ref, out_ref)


def _grouped_gemm_half(sm, se, lo, hi, xs_h, w, htile0, ys_prev):
    in_specs = [
        pl.BlockSpec((_BM, _INTER), lambda i, sm, se, lo, hi: (sm[i], 0)),
        pl.BlockSpec((None, _INTER, _HID),
                     lambda i, sm, se, lo, hi: (se[i], 0, 0)),
    ]
    args = [sm, se, lo, hi, xs_h, w]
    body = _gemm_body
    aliases = {}
    if ys_prev is not None:
        in_specs.append(pl.BlockSpec(memory_space=pl.ANY))
        args.append(ys_prev)
        body = _gemm_body_alias
        aliases = {6: 0}
    grid_spec = pltpu.PrefetchScalarGridSpec(
        num_scalar_prefetch=4,
        grid=(_MAX_STEPS_H,),
        in_specs=in_specs,
        out_specs=pl.BlockSpec((_BM, _HID),
                               lambda i, sm, se, lo, hi: (htile0 + sm[i], 0)),
    )
    return pl.pallas_call(
        body,
        grid_spec=grid_spec,
        out_shape=jax.ShapeDtypeStruct((_NSLOT, _HID), jnp.bfloat16),
        input_output_aliases=aliases,
        compiler_params=pltpu.CompilerParams(
            dimension_semantics=("arbitrary",)),
    )(*args)


_BM2 = 512


def _pair_add_body(y2_ref, ew_ref, out_ref):
    out_ref[...] = (y2_ref[:, :_HID].astype(jnp.float32) * ew_ref[:, 0:1]
                    + y2_ref[:, _HID:].astype(jnp.float32) * ew_ref[:, 1:2])


def _pair_add(y2, ew):
    return pl.pallas_call(
        _pair_add_body,
        grid=(_NTOK // _BM2,),
        in_specs=[
            pl.BlockSpec((_BM2, _TOPK * _HID), lambda i: (i, 0)),
            pl.BlockSpec((_BM2, _TOPK), lambda i: (i, 0)),
        ],
        out_specs=pl.BlockSpec((_BM2, _HID), lambda i: (i, 0)),
        out_shape=jax.ShapeDtypeStruct((_NTOK, _HID), jnp.float32),
    )(y2, ew)


def _route_metadata(flat, r0):
    counts = jnp.bincount(flat, length=_E).astype(jnp.int32)
    offs = jnp.concatenate(
        [jnp.zeros((1,), jnp.int32), jnp.cumsum(counts).astype(jnp.int32)])
    offs = jnp.clip(offs, r0, r0 + _HROWS) - r0
    counts_w = offs[1:] - offs[:_E]
    tile_first = offs[:_E] // _BM
    steps_e = jnp.where(
        counts_w > 0, (offs[1:] + _BM - 1) // _BM - tile_first, 0)
    estart = jnp.concatenate(
        [jnp.zeros((1,), jnp.int32), jnp.cumsum(steps_e).astype(jnp.int32)])
    total = estart[_E]
    i = jnp.arange(_MAX_STEPS_H, dtype=jnp.int32)
    e_of = jnp.clip(
        jnp.searchsorted(estart, i, side="right").astype(jnp.int32) - 1,
        0, _E - 1)
    valid = i < total
    e_last = (jnp.searchsorted(estart, total - 1, side="right")
              .astype(jnp.int32) - 1)
    e_of = jnp.where(valid, e_of, e_last)
    m_of = jnp.where(valid, tile_first[e_of] + (i - estart[e_of]),
                     _HTILES - 1)
    lo_abs = jnp.maximum(offs[e_of], m_of * _BM)
    hi_abs = jnp.minimum(offs[e_of + 1], (m_of + 1) * _BM)
    lo = jnp.where(valid, lo_abs - m_of * _BM, 0)
    hi = jnp.where(valid, hi_abs - m_of * _BM, 0)
    return m_of, e_of, lo, hi


def kernel(x, w, chosen_experts, expert_weight):
    flat = chosen_experts.reshape(-1).astype(jnp.int32)
    perm = jnp.argsort(flat).astype(jnp.int32)
    inv = jnp.zeros((_NSLOT,), jnp.int32).at[perm].set(
        jnp.arange(_NSLOT, dtype=jnp.int32))

    ys = None
    for h in range(_NSPLIT):
        r0 = h * _HROWS
        xs_h = _sc_gather_xh()(x, lax.dynamic_slice_in_dim(perm, r0, _HROWS))
        sm, se, lo, hi = _route_metadata(flat, r0)
        ys = _grouped_gemm_half(sm, se, lo, hi, xs_h, w, r0 // _BM, ys)

    ys_bits = lax.bitcast_convert_type(
        ys.reshape(_NSLOT, _HID // 2, 2), jnp.int32)
    ys_slot = lax.bitcast_convert_type(
        _sc_unsort_y()(ys_bits, inv), jnp.bfloat16)
    out = _pair_add(ys_slot.reshape(_NTOK, _TOPK * _HID),
                    expert_weight.astype(jnp.float32))
    return out


# speedup vs baseline: 15.0925x; 15.0925x over previous
import functools

import jax
import jax.numpy as jnp
from jax import lax
from jax.experimental import pallas as pl
from jax.experimental.pallas import tpu as pltpu
from jax.experimental.pallas import tpu_sc as plsc

_NTOK = 8192
_TOPK = 2
_E = 8
_INTER = 4096
_HID = 1024
_NSLOT = _NTOK * _TOPK

_BM = 512
_NTILES = _NSLOT // _BM
_MAX_STEPS = _NTILES + _E - 1

_NC, _NS = 2, 16
_NW = _NC * _NS


def _make_ring_gather(nrows, width, ch, dtype=jnp.float32):
    rpw = nrows // _NW
    nch = rpw // ch
    assert rpw % ch == 0 and nch % 2 == 0

    def body(src_hbm, idx_hbm, out_hbm, idx_v, buf0, buf1,
             gsem0, gsem1, ssem0, ssem1):
        bufs, gsems, ssems = (buf0, buf1), (gsem0, gsem1), (ssem0, ssem1)
        wid = lax.axis_index("s") * _NC + lax.axis_index("c")
        base = wid * rpw
        pltpu.sync_copy(idx_hbm.at[pl.ds(base, rpw)], idx_v)

        def gather(c, b):
            pltpu.make_async_copy(
                src_hbm.at[idx_v.at[pl.ds(c * ch, ch)]], bufs[b], gsems[b]
            ).start()

        gather(0, 0)
        gather(1, 1)

        def outer(c0, carry):
            for b in (0, 1):
                c = 2 * c0 + b
                pltpu.make_async_copy(
                    src_hbm.at[idx_v.at[pl.ds(c * ch, ch)]], bufs[b], gsems[b]
                ).wait()
                scat = pltpu.make_async_copy(
                    bufs[b], out_hbm.at[pl.ds(base + c * ch, ch)], ssems[b])
                scat.start()
                scat.wait()

                @pl.when(c + 2 < nch)
                def _():
                    gather(c + 2, b)
            return carry

        lax.fori_loop(0, nch // 2, outer, 0)

    @functools.cache
    def build():
        return pl.kernel(
            body,
            mesh=plsc.VectorSubcoreMesh(
                core_axis_name="c", subcore_axis_name="s"),
            out_type=jax.ShapeDtypeStruct((nrows, width), dtype),
            scratch_types=[
                pltpu.VMEM((rpw,), jnp.int32),
                pltpu.VMEM((ch, width), dtype),
                pltpu.VMEM((ch, width), dtype),
                pltpu.SemaphoreType.DMA,
                pltpu.SemaphoreType.DMA,
                pltpu.SemaphoreType.DMA,
                pltpu.SemaphoreType.DMA,
            ],
        )

    return build


_NSPLIT = 1
_HROWS = _NSLOT // _NSPLIT
_HTILES = _HROWS // _BM
_MAX_STEPS_H = _HTILES + _E - 1

_sc_gather_xh = _make_ring_gather(_HROWS, _INTER, 8)
_sc_unsort_y = _make_ring_gather(_NSLOT, _HID, 32)


def _gemm_body(sm_ref, se_ref, lo_ref, hi_ref, xs_ref, w_ref, out_ref):
    i = pl.program_id(0)
    acc = jnp.dot(xs_ref[...].astype(jnp.bfloat16),
                  w_ref[...].astype(jnp.bfloat16),
                  preferred_element_type=jnp.float32)
    rows = lax.broadcasted_iota(jnp.int32, (_BM, _HID), 0)
    mask = (rows >= lo_ref[i]) & (rows < hi_ref[i])
    out_ref[...] = jnp.where(mask, acc, out_ref[...])


def _gemm_body_alias(sm_ref, se_ref, lo_ref, hi_ref, xs_ref, w_ref, prev_ref,
                     out_ref):
    del prev_ref
    _gemm_body(sm_ref, se_ref, lo_ref, hi_ref, xs_ref, w_ref, out_ref)


def _grouped_gemm_half(sm, se, lo, hi, xs_h, w, htile0, ys_prev):
    in_specs = [
        pl.BlockSpec((_BM, _INTER), lambda i, sm, se, lo, hi: (sm[i], 0)),
        pl.BlockSpec((None, _INTER, _HID),
                     lambda i, sm, se, lo, hi: (se[i], 0, 0)),
    ]
    args = [sm, se, lo, hi, xs_h, w]
    body = _gemm_body
    aliases = {}
    if ys_prev is not None:
        in_specs.append(pl.BlockSpec(memory_space=pl.ANY))
        args.append(ys_prev)
        body = _gemm_body_alias
        aliases = {6: 0}
    grid_spec = pltpu.PrefetchScalarGridSpec(
        num_scalar_prefetch=4,
        grid=(_MAX_STEPS_H,),
        in_specs=in_specs,
        out_specs=pl.BlockSpec((_BM, _HID),
                               lambda i, sm, se, lo, hi: (htile0 + sm[i], 0)),
    )
    return pl.pallas_call(
        body,
        grid_spec=grid_spec,
        out_shape=jax.ShapeDtypeStruct((_NSLOT, _HID), jnp.float32),
        input_output_aliases=aliases,
        compiler_params=pltpu.CompilerParams(
            dimension_semantics=("arbitrary",)),
    )(*args)


_BM2 = 512


def _pair_add_body(y2_ref, ew_ref, out_ref):
    out_ref[...] = (y2_ref[:, :_HID] * ew_ref[:, 0:1]
                    + y2_ref[:, _HID:] * ew_ref[:, 1:2])


def _pair_add(y2, ew):
    return pl.pallas_call(
        _pair_add_body,
        grid=(_NTOK // _BM2,),
        in_specs=[
            pl.BlockSpec((_BM2, _TOPK * _HID), lambda i: (i, 0)),
            pl.BlockSpec((_BM2, _TOPK), lambda i: (i, 0)),
        ],
        out_specs=pl.BlockSpec((_BM2, _HID), lambda i: (i, 0)),
        out_shape=jax.ShapeDtypeStruct((_NTOK, _HID), jnp.float32),
    )(y2, ew)


def _route_metadata(flat, r0):
    counts = jnp.bincount(flat, length=_E).astype(jnp.int32)
    offs = jnp.concatenate(
        [jnp.zeros((1,), jnp.int32), jnp.cumsum(counts).astype(jnp.int32)])
    offs = jnp.clip(offs, r0, r0 + _HROWS) - r0
    counts_w = offs[1:] - offs[:_E]
    tile_first = offs[:_E] // _BM
    steps_e = jnp.where(
        counts_w > 0, (offs[1:] + _BM - 1) // _BM - tile_first, 0)
    estart = jnp.concatenate(
        [jnp.zeros((1,), jnp.int32), jnp.cumsum(steps_e).astype(jnp.int32)])
    total = estart[_E]
    i = jnp.arange(_MAX_STEPS_H, dtype=jnp.int32)
    e_of = jnp.clip(
        jnp.searchsorted(estart, i, side="right").astype(jnp.int32) - 1,
        0, _E - 1)
    valid = i < total
    e_last = (jnp.searchsorted(estart, total - 1, side="right")
              .astype(jnp.int32) - 1)
    e_of = jnp.where(valid, e_of, e_last)
    m_of = jnp.where(valid, tile_first[e_of] + (i - estart[e_of]),
                     _HTILES - 1)
    lo_abs = jnp.maximum(offs[e_of], m_of * _BM)
    hi_abs = jnp.minimum(offs[e_of + 1], (m_of + 1) * _BM)
    lo = jnp.where(valid, lo_abs - m_of * _BM, 0)
    hi = jnp.where(valid, hi_abs - m_of * _BM, 0)
    return m_of, e_of, lo, hi


def kernel(x, w, chosen_experts, expert_weight):
    flat = chosen_experts.reshape(-1).astype(jnp.int32)
    perm = jnp.argsort(flat).astype(jnp.int32)
    inv = jnp.zeros((_NSLOT,), jnp.int32).at[perm].set(
        jnp.arange(_NSLOT, dtype=jnp.int32))

    ys = None
    for h in range(_NSPLIT):
        r0 = h * _HROWS
        xs_h = _sc_gather_xh()(x, lax.dynamic_slice_in_dim(perm, r0, _HROWS))
        sm, se, lo, hi = _route_metadata(flat, r0)
        ys = _grouped_gemm_half(sm, se, lo, hi, xs_h, w, r0 // _BM, ys)

    ys_slot = _sc_unsort_y()(ys, inv)
    out = _pair_add(ys_slot.reshape(_NTOK, _TOPK * _HID),
                    expert_weight.astype(jnp.float32))
    return out
